# baseline (speedup 1.0000x reference)
import functools

import jax
import jax.numpy as jnp
from jax import lax
from jax.experimental import pallas as pl
from jax.experimental.pallas import tpu as pltpu
from jax.experimental.pallas import tpu_sc as plsc

NC = 2
NS = 16
NW = NC * NS

NBUF = 14
CHUNK = 8


@functools.lru_cache(maxsize=None)
def _make_gather(V, D, R, S):
    assert R % NW == 0
    cpw = R // NW
    assert cpw % CHUNK == 0
    cpj = cpw // CHUNK
    n_steps = S * cpj
    assert n_steps >= 2 * NBUF
    n_rounds = n_steps // NBUF

    mesh = plsc.VectorSubcoreMesh(core_axis_name="c", subcore_axis_name="s")

    @functools.partial(
        pl.kernel,
        mesh=mesh,
        out_type=jax.ShapeDtypeStruct((S * R, D), jnp.float32),
        scratch_types=[
            pltpu.VMEM((S * cpw,), jnp.int32),
            pltpu.VMEM((NBUF, CHUNK, D), jnp.float32),
        ] + [pltpu.SemaphoreType.DMA] * (2 * NBUF),
    )
    def gather_kernel(idx_hbm, table_hbm, out_hbm, idx_v, rows_v, *sems):
        gsem = sems[:NBUF]
        ssem = sems[NBUF:]
        wid = lax.axis_index("s") * NC + lax.axis_index("c")
        base = wid * cpw
        pltpu.sync_copy(idx_hbm.at[wid], idx_v)

        def gather_copy(t, b):
            return pltpu.make_async_copy(
                table_hbm.at[idx_v.at[pl.ds(t * CHUNK, CHUNK)]],
                rows_v.at[b],
                gsem[b],
            )

        def store_copy(t, b):
            j = t // cpj
            part = t % cpj
            return pltpu.make_async_copy(
                rows_v.at[b],
                out_hbm.at[pl.ds(j * R + base + part * CHUNK, CHUNK)],
                ssem[b],
            )

        for b in range(NBUF):
            gather_copy(b, b).start()

        def round_body(g, carry):
            for b in range(NBUF):
                t = g * NBUF + b
                gather_copy(t, b).wait()
                store_copy(t, b).start()
            for b in range(NBUF):
                t = g * NBUF + b
                store_copy(t, b).wait()
                gather_copy(t + NBUF, b).start()
            return carry

        lax.fori_loop(0, n_rounds - 1, round_body, 0)

        for t in range((n_rounds - 1) * NBUF, n_steps):
            b = t % NBUF
            gather_copy(t, b).wait()
            store_copy(t, b).start()
            if t + NBUF < n_steps:
                store_copy(t, b).wait()
                gather_copy(t + NBUF, b).start()
        for t in range(n_steps - NBUF, n_steps):
            store_copy(t, t % NBUF).wait()

    return gather_kernel


@jax.jit
def _embed(x, table):
    V, D = table.shape
    R, S = x.shape
    cpw = R // NW
    idx_blocks = (
        x.astype(jnp.int32).T.reshape(S, NW, cpw).swapaxes(0, 1).reshape(NW, -1)
    )
    out = _make_gather(V, D, R, S)(idx_blocks, table)
    return out.reshape(S, R, D).transpose(1, 0, 2)


def kernel(x, table):
    return _embed(x, table)


# speedup vs baseline: 1.0220x; 1.0220x over previous
import functools

import jax
import jax.numpy as jnp
from jax import lax
from jax.experimental import pallas as pl
from jax.experimental.pallas import tpu as pltpu
from jax.experimental.pallas import tpu_sc as plsc

NC = 2
NS = 16
NW = NC * NS

NBUF = 10
CHUNK = 8


@functools.lru_cache(maxsize=None)
def _make_gather(V, D, R, S):
    assert R % NW == 0
    cpw = R // NW
    assert cpw % CHUNK == 0
    cpj = cpw // CHUNK
    n_steps = S * cpj
    assert n_steps >= 2 * NBUF
    n_rounds = n_steps // NBUF

    mesh = plsc.VectorSubcoreMesh(core_axis_name="c", subcore_axis_name="s")

    @functools.partial(
        pl.kernel,
        mesh=mesh,
        out_type=jax.ShapeDtypeStruct((S * R, D), jnp.float32),
        scratch_types=[
            pltpu.VMEM((S * cpw,), jnp.int32),
            pltpu.VMEM((NBUF, CHUNK, D), jnp.float32),
        ] + [pltpu.SemaphoreType.DMA] * (2 * NBUF),
    )
    def gather_kernel(idx_hbm, table_hbm, out_hbm, idx_v, rows_v, *sems):
        gsem = sems[:NBUF]
        ssem = sems[NBUF:]
        wid = lax.axis_index("s") * NC + lax.axis_index("c")
        base = wid * cpw
        pltpu.sync_copy(idx_hbm.at[wid], idx_v)

        def gather_copy(t, b):
            return pltpu.make_async_copy(
                table_hbm.at[idx_v.at[pl.ds(t * CHUNK, CHUNK)]],
                rows_v.at[b],
                gsem[b],
            )

        def store_copy(t, b):
            j = t // cpj
            part = t % cpj
            return pltpu.make_async_copy(
                rows_v.at[b],
                out_hbm.at[pl.ds(j * R + base + part * CHUNK, CHUNK)],
                ssem[b],
            )

        for b in range(NBUF):
            gather_copy(b, b).start()

        def round_body(g, carry):
            for b in range(NBUF):
                t = g * NBUF + b
                gather_copy(t, b).wait()
                store_copy(t, b).start()
            for b in range(NBUF):
                t = g * NBUF + b
                store_copy(t, b).wait()
                gather_copy(t + NBUF, b).start()
            return carry

        lax.fori_loop(0, n_rounds - 1, round_body, 0)

        for t in range((n_rounds - 1) * NBUF, n_steps):
            b = t % NBUF
            gather_copy(t, b).wait()
            store_copy(t, b).start()
            if t + NBUF < n_steps:
                store_copy(t, b).wait()
                gather_copy(t + NBUF, b).start()
        for t in range(n_steps - NBUF, n_steps):
            store_copy(t, t % NBUF).wait()

    return gather_kernel


@jax.jit
def _embed(x, table):
    V, D = table.shape
    R, S = x.shape
    cpw = R // NW
    idx_blocks = (
        x.astype(jnp.int32).T.reshape(S, NW, cpw).swapaxes(0, 1).reshape(NW, -1)
    )
    out = _make_gather(V, D, R, S)(idx_blocks, table)
    return out.reshape(S, R, D).transpose(1, 0, 2)


def kernel(x, table):
    return _embed(x, table)
